# Initial kernel scaffold; baseline (speedup 1.0000x reference)
import jax
import jax.numpy as jnp
from jax.experimental import pallas as pl


def kernel(fea, adj, W1, b1, W2, b2):
    raise NotImplementedError("write your pallas kernel here")


# speedup vs baseline: 9.1522x; 9.1522x over previous
import functools

import jax
import jax.numpy as jnp
from jax import lax
from jax.experimental import pallas as pl
from jax.experimental.pallas import tpu as pltpu
from jax.experimental.pallas import tpu_sc as plsc

N = 10000
E = 320000
NC, NS = 2, 16
NW = NC * NS
CHUNK = 128
NCH = -(-E // (NW * CHUNK))
EPT = NCH * CHUNK
E_PAD = EPT * NW
ACC_ROWS = 10240
TRASH = ACC_ROWS - N
ZROWS = ACC_ROWS // NS
OROWS = ACC_ROWS // NS

_mesh = plsc.VectorSubcoreMesh(
    core_axis_name="c", subcore_axis_name="s", num_cores=NC, num_subcores=NS
)


def _make_sc_aggregate(D: int):
  def body(support, src_i, dst_i, out, acc, isrc, idst, rows, sem):
    c = lax.axis_index("c")
    s = lax.axis_index("s")
    wid = c * NS + s

    pltpu.sync_copy(src_i.at[wid], isrc)
    pltpu.sync_copy(dst_i.at[wid], idst)

    def zrow(i, _):
      for k in range(D // 16):
        rows[i, pl.ds(k * 16, 16)] = jnp.zeros((16,), jnp.float32)
      return 0

    lax.fori_loop(0, CHUNK, zrow, 0)
    base = s * ZROWS
    off = 0
    while off < ZROWS:
      sz = min(CHUNK, ZROWS - off)
      pltpu.sync_copy(rows.at[pl.ds(0, sz)], acc.at[pl.ds(base + off, sz)])
      off += sz
    plsc.subcore_barrier()

    def step(j, _):
      pltpu.async_copy(support.at[isrc.at[j]], rows, sem).wait()
      pltpu.sync_copy(rows, acc.at[idst.at[j]], add=True)
      return 0

    lax.fori_loop(0, NCH, step, 0)
    plsc.subcore_barrier()

    ob = s * OROWS
    pltpu.sync_copy(acc.at[pl.ds(ob, OROWS)], out.at[c].at[pl.ds(ob, OROWS)])

  return pl.kernel(
      body,
      out_type=jax.ShapeDtypeStruct((NC, ACC_ROWS, D), jnp.float32),
      mesh=_mesh,
      scratch_types=[
          pltpu.VMEM_SHARED((ACC_ROWS, D), jnp.float32),
          pltpu.VMEM((NCH, CHUNK), jnp.int32),
          pltpu.VMEM((NCH, CHUNK), jnp.int32),
          pltpu.VMEM((CHUNK, D), jnp.float32),
          pltpu.SemaphoreType.DMA,
      ],
      compiler_params=pltpu.CompilerParams(use_tc_tiling_on_sc=False),
  )


_sc_agg_128 = _make_sc_aggregate(128)
_sc_agg_64 = _make_sc_aggregate(64)


def _mm_bias_kernel(x_ref, w_ref, b_ref, o_ref):
  o_ref[...] = (
      jnp.dot(x_ref[...], w_ref[...], preferred_element_type=jnp.float32)
      + b_ref[...]
  )


def _mm_bias_sum2_kernel(p_ref, w_ref, b_ref, o_ref):
  x = p_ref[0] + p_ref[1]
  o_ref[...] = (
      jnp.dot(x, w_ref[...], preferred_element_type=jnp.float32) + b_ref[...]
  )


def _sum2_kernel(p_ref, o_ref):
  o_ref[...] = p_ref[0] + p_ref[1]


_BN = 1000


def _tc_mm_bias(x, w, b):
  din, dout = w.shape
  return pl.pallas_call(
      _mm_bias_kernel,
      grid=(N // _BN,),
      in_specs=[
          pl.BlockSpec((_BN, din), lambda i: (i, 0)),
          pl.BlockSpec((din, dout), lambda i: (0, 0)),
          pl.BlockSpec((1, dout), lambda i: (0, 0)),
      ],
      out_specs=pl.BlockSpec((_BN, dout), lambda i: (i, 0)),
      out_shape=jax.ShapeDtypeStruct((N, dout), jnp.float32),
  )(x, w, b.reshape(1, dout))


def _tc_mm_bias_sum2(p, w, b):
  din, dout = w.shape
  return pl.pallas_call(
      _mm_bias_sum2_kernel,
      grid=(N // _BN,),
      in_specs=[
          pl.BlockSpec((2, _BN, din), lambda i: (0, i, 0)),
          pl.BlockSpec((din, dout), lambda i: (0, 0)),
          pl.BlockSpec((1, dout), lambda i: (0, 0)),
      ],
      out_specs=pl.BlockSpec((_BN, dout), lambda i: (i, 0)),
      out_shape=jax.ShapeDtypeStruct((N, dout), jnp.float32),
  )(p, w, b.reshape(1, dout))


def _tc_sum2(p):
  d = p.shape[-1]
  return pl.pallas_call(
      _sum2_kernel,
      grid=(N // _BN,),
      in_specs=[pl.BlockSpec((2, _BN, d), lambda i: (0, i, 0))],
      out_specs=pl.BlockSpec((_BN, d), lambda i: (i, 0)),
      out_shape=jax.ShapeDtypeStruct((N, d), jnp.float32),
  )(p)


@jax.jit
def kernel(fea, adj, W1, b1, W2, b2):
  adj = adj.astype(jnp.int32)
  src = adj[0]
  dst = adj[1]
  pad = E_PAD - E
  fill = jnp.arange(pad, dtype=jnp.int32)
  src_p = jnp.concatenate([src, fill % N]).reshape(NW, NCH, CHUNK)
  dst_p = jnp.concatenate([dst, N + (fill % TRASH)]).reshape(NW, NCH, CHUNK)

  s1 = _tc_mm_bias(fea, W1, b1)
  p1 = _sc_agg_128(s1, src_p, dst_p)
  s2 = _tc_mm_bias_sum2(p1, W2, b2)
  p2 = _sc_agg_64(s2, src_p, dst_p)
  return _tc_sum2(p2)


# speedup vs baseline: 11.2967x; 1.2343x over previous
import functools

import jax
import jax.numpy as jnp
from jax import lax
from jax.experimental import pallas as pl
from jax.experimental.pallas import tpu as pltpu
from jax.experimental.pallas import tpu_sc as plsc

N = 10000
E = 320000
NC, NS = 2, 16
NW = NC * NS
CHUNK = 64
NCH = 158
EPT = NCH * CHUNK
E_PAD = EPT * NW
ACC_ROWS = 10240
TRASH = ACC_ROWS - N
ZROWS = ACC_ROWS // NS
OROWS = ACC_ROWS // NS

_mesh = plsc.VectorSubcoreMesh(
    core_axis_name="c", subcore_axis_name="s", num_cores=NC, num_subcores=NS
)


def _make_sc_aggregate(D: int):
  def body(support, src_i, dst_i, out, acc, isrc, idst, rows0, rows1, sem0,
           sem1):
    c = lax.axis_index("c")
    s = lax.axis_index("s")
    wid = c * NS + s

    pltpu.sync_copy(src_i.at[wid], isrc)
    pltpu.sync_copy(dst_i.at[wid], idst)

    def zrow(i, _):
      for k in range(D // 16):
        rows0[i, pl.ds(k * 16, 16)] = jnp.zeros((16,), jnp.float32)
      return 0

    lax.fori_loop(0, CHUNK, zrow, 0)
    base = s * ZROWS
    off = 0
    while off < ZROWS:
      sz = min(CHUNK, ZROWS - off)
      pltpu.sync_copy(rows0.at[pl.ds(0, sz)], acc.at[pl.ds(base + off, sz)])
      off += sz
    plsc.subcore_barrier()

    pltpu.async_copy(support.at[isrc.at[0]], rows0, sem0)

    def step(h, _):
      j = 2 * h
      pltpu.async_copy(support.at[isrc.at[j + 1]], rows1, sem1)
      pltpu.make_async_copy(support.at[isrc.at[j]], rows0, sem0).wait()
      pltpu.sync_copy(rows0, acc.at[idst.at[j]], add=True)

      @pl.when(j + 2 < NCH)
      def _():
        pltpu.async_copy(support.at[isrc.at[j + 2]], rows0, sem0)

      pltpu.make_async_copy(support.at[isrc.at[j + 1]], rows1, sem1).wait()
      pltpu.sync_copy(rows1, acc.at[idst.at[j + 1]], add=True)
      return 0

    lax.fori_loop(0, NCH // 2, step, 0)
    plsc.subcore_barrier()

    ob = s * OROWS
    pltpu.sync_copy(acc.at[pl.ds(ob, OROWS)], out.at[c].at[pl.ds(ob, OROWS)])

  return pl.kernel(
      body,
      out_type=jax.ShapeDtypeStruct((NC, ACC_ROWS, D), jnp.float32),
      mesh=_mesh,
      scratch_types=[
          pltpu.VMEM_SHARED((ACC_ROWS, D), jnp.float32),
          pltpu.VMEM((NCH, CHUNK), jnp.int32),
          pltpu.VMEM((NCH, CHUNK), jnp.int32),
          pltpu.VMEM((CHUNK, D), jnp.float32),
          pltpu.VMEM((CHUNK, D), jnp.float32),
          pltpu.SemaphoreType.DMA,
          pltpu.SemaphoreType.DMA,
      ],
      compiler_params=pltpu.CompilerParams(use_tc_tiling_on_sc=False),
  )


_sc_agg_128 = _make_sc_aggregate(128)
_sc_agg_64 = _make_sc_aggregate(64)


def _mm_bias_kernel(x_ref, w_ref, b_ref, o_ref):
  o_ref[...] = (
      jnp.dot(x_ref[...], w_ref[...], preferred_element_type=jnp.float32)
      + b_ref[...]
  )


def _mm_bias_sum2_kernel(p_ref, w_ref, b_ref, o_ref):
  x = p_ref[0] + p_ref[1]
  o_ref[...] = (
      jnp.dot(x, w_ref[...], preferred_element_type=jnp.float32) + b_ref[...]
  )


def _sum2_kernel(p_ref, o_ref):
  o_ref[...] = p_ref[0] + p_ref[1]


_BN = 1000


def _tc_mm_bias(x, w, b):
  din, dout = w.shape
  return pl.pallas_call(
      _mm_bias_kernel,
      grid=(N // _BN,),
      in_specs=[
          pl.BlockSpec((_BN, din), lambda i: (i, 0)),
          pl.BlockSpec((din, dout), lambda i: (0, 0)),
          pl.BlockSpec((1, dout), lambda i: (0, 0)),
      ],
      out_specs=pl.BlockSpec((_BN, dout), lambda i: (i, 0)),
      out_shape=jax.ShapeDtypeStruct((N, dout), jnp.float32),
  )(x, w, b.reshape(1, dout))


def _tc_mm_bias_sum2(p, w, b):
  din, dout = w.shape
  return pl.pallas_call(
      _mm_bias_sum2_kernel,
      grid=(N // _BN,),
      in_specs=[
          pl.BlockSpec((2, _BN, din), lambda i: (0, i, 0)),
          pl.BlockSpec((din, dout), lambda i: (0, 0)),
          pl.BlockSpec((1, dout), lambda i: (0, 0)),
      ],
      out_specs=pl.BlockSpec((_BN, dout), lambda i: (i, 0)),
      out_shape=jax.ShapeDtypeStruct((N, dout), jnp.float32),
  )(p, w, b.reshape(1, dout))


def _tc_sum2(p):
  d = p.shape[-1]
  return pl.pallas_call(
      _sum2_kernel,
      grid=(N // _BN,),
      in_specs=[pl.BlockSpec((2, _BN, d), lambda i: (0, i, 0))],
      out_specs=pl.BlockSpec((_BN, d), lambda i: (i, 0)),
      out_shape=jax.ShapeDtypeStruct((N, d), jnp.float32),
  )(p)


@jax.jit
def kernel(fea, adj, W1, b1, W2, b2):
  adj = adj.astype(jnp.int32)
  src = adj[0]
  dst = adj[1]
  pad = E_PAD - E
  fill = jnp.arange(pad, dtype=jnp.int32)
  src_p = jnp.concatenate([src, fill % N]).reshape(NW, NCH, CHUNK)
  dst_p = jnp.concatenate([dst, N + (fill % TRASH)]).reshape(NW, NCH, CHUNK)

  s1 = _tc_mm_bias(fea, W1, b1)
  p1 = _sc_agg_128(s1, src_p, dst_p)
  s2 = _tc_mm_bias_sum2(p1, W2, b2)
  p2 = _sc_agg_64(s2, src_p, dst_p)
  return _tc_sum2(p2)
